# Initial kernel scaffold; baseline (speedup 1.0000x reference)
import functools

import jax
import jax.numpy as jnp
from jax import lax
from jax.experimental import pallas as pl
from jax.experimental.pallas import tpu as pltpu
from jax.experimental.pallas import tpu_sc as plsc

_NC = 2
_NS = 16
_L = 16
_CH = 80
_ZR = 128


def _sc_segment_sums(nodes, edges, senders, receivers):
    n, d = nodes.shape
    e, de = edges.shape
    ept = e // (_NC * _NS)
    nch = ept // _CH
    rpt = n // _NS

    mesh = plsc.VectorSubcoreMesh(core_axis_name="c", subcore_axis_name="s")

    @functools.partial(
        pl.kernel,
        out_type=(
            jax.ShapeDtypeStruct((_NC * n, d), jnp.float32),
            jax.ShapeDtypeStruct((_NC * n, de), jnp.float32),
        ),
        mesh=mesh,
        scratch_types=[
            pltpu.VMEM((_CH,), jnp.int32),
            pltpu.VMEM((_CH,), jnp.int32),
            pltpu.VMEM((_CH, d), jnp.float32),
            pltpu.VMEM((_CH, de), jnp.float32),
            pltpu.VMEM((_ZR, d), jnp.float32),
            pltpu.VMEM((n // _NS, de), jnp.float32),
            pltpu.VMEM_SHARED((n, d), jnp.float32),
            pltpu.VMEM_SHARED((n, de), jnp.float32),
            pltpu.SemaphoreType.DMA,
        ],
    )
    def seg(nodes_h, edges_h, send_h, recv_h, gout_h, eout_h,
            idx_s, idx_r, rows, erows, zn, ze, g_sh, e_sh, sem):
        c = lax.axis_index("c")
        s = lax.axis_index("s")
        zero16 = jnp.zeros((_L,), jnp.float32)
        dsub = d // _L

        @pl.loop(0, _ZR * dsub)
        def _(i):
            zn[i // dsub, pl.ds((i % dsub) * _L, _L)] = zero16

        @pl.loop(0, rpt)
        def _(i):
            ze[i, :] = zero16

        row0 = s * rpt
        nfull = rpt // _ZR

        @pl.loop(0, nfull)
        def _(i):
            pltpu.sync_copy(zn, g_sh.at[pl.ds(row0 + i * _ZR, _ZR)])

        rem = rpt - nfull * _ZR
        if rem:
            pltpu.sync_copy(zn.at[pl.ds(0, rem)],
                            g_sh.at[pl.ds(row0 + nfull * _ZR, rem)])
        pltpu.sync_copy(ze, e_sh.at[pl.ds(row0, rpt)])

        plsc.subcore_barrier()

        base = (c * _NS + s) * ept

        @pl.loop(0, nch)
        def _(k):
            off = pl.multiple_of(base + k * _CH, 8)
            pltpu.sync_copy(send_h.at[pl.ds(off, _CH)], idx_s)
            pltpu.sync_copy(recv_h.at[pl.ds(off, _CH)], idx_r)
            pltpu.async_copy(nodes_h.at[idx_s], rows, sem).wait()
            pltpu.sync_copy(edges_h.at[pl.ds(off, _CH)], erows)
            pltpu.sync_copy(rows, g_sh.at[idx_r], add=True)
            pltpu.sync_copy(erows, e_sh.at[idx_r], add=True)

        plsc.subcore_barrier()

        out0 = c * n + row0
        pltpu.sync_copy(g_sh.at[pl.ds(row0, rpt)], gout_h.at[pl.ds(out0, rpt)])
        pltpu.sync_copy(e_sh.at[pl.ds(row0, rpt)], eout_h.at[pl.ds(out0, rpt)])

    return seg(nodes, edges, senders, receivers)


def _dense_body(g_ref, e_ref, x_ref, a_ref, b_ref, w1_ref, b1_ref,
                w2_ref, b2_ref, s1_ref, o1_ref, s2_ref, o2_ref, out_ref):
    hi = lax.Precision.HIGHEST
    g = g_ref[0] + g_ref[1]
    ea = e_ref[0] + e_ref[1]
    x = x_ref[...]
    agg = (jnp.dot(g, a_ref[...], precision=hi)
           + jnp.dot(ea, b_ref[...], precision=hi))
    agg = jnp.maximum(agg, 0.0) + x
    mean = jnp.mean(agg, axis=-1, keepdims=True)
    ctr = agg - mean
    var = jnp.mean(ctr * ctr, axis=-1, keepdims=True)
    aggn = ctr * lax.rsqrt(var + 1e-6) * s1_ref[...] + o1_ref[...]
    h = jnp.maximum(jnp.dot(aggn, w1_ref[...], precision=hi) + b1_ref[...], 0.0)
    y = jnp.dot(h, w2_ref[...], precision=hi) + b2_ref[...] + aggn
    t = x + y
    mean2 = jnp.mean(t, axis=-1, keepdims=True)
    ctr2 = t - mean2
    var2 = jnp.mean(ctr2 * ctr2, axis=-1, keepdims=True)
    out_ref[...] = ctr2 * lax.rsqrt(var2 + 1e-6) * s2_ref[...] + o2_ref[...]


def _tc_dense(gpart, epart, nodes, a, b, w1, b1, w2, b2, s1, o1, s2, o2):
    n, d = nodes.shape
    de = epart.shape[-1]
    bn = 1250
    grid = n // bn
    full = lambda shape: pl.BlockSpec(shape, lambda i, _s=len(shape): (0,) * _s)
    return pl.pallas_call(
        _dense_body,
        grid=(grid,),
        in_specs=[
            pl.BlockSpec((_NC, bn, d), lambda i: (0, i, 0)),
            pl.BlockSpec((_NC, bn, de), lambda i: (0, i, 0)),
            pl.BlockSpec((bn, d), lambda i: (i, 0)),
            full((d, d)), full((de, d)), full((d, d)), full((1, d)),
            full((d, d)), full((1, d)), full((1, d)), full((1, d)),
            full((1, d)), full((1, d)),
        ],
        out_specs=pl.BlockSpec((bn, d), lambda i: (i, 0)),
        out_shape=jax.ShapeDtypeStruct((n, d), jnp.float32),
    )(gpart, epart, nodes, a, b, w1, b1, w2, b2, s1, o1, s2, o2)


def kernel(nodes, edges, receivers, senders, W_msg, W1, b1, W2, b2,
           ln_aggr_scale, ln_aggr_bias, ln_out_scale, ln_out_bias):
    n, d = nodes.shape
    de = edges.shape[1]
    gpart, epart = _sc_segment_sums(nodes, edges, senders, receivers)
    r = lambda v: v.reshape(1, d)
    return _tc_dense(
        gpart.reshape(_NC, n, d), epart.reshape(_NC, n, de), nodes,
        W_msg[:d], W_msg[d:], W1, r(b1), W2, r(b2),
        r(ln_aggr_scale), r(ln_aggr_bias), r(ln_out_scale), r(ln_out_bias))


# speedup vs baseline: 2.6947x; 2.6947x over previous
import functools

import jax
import jax.numpy as jnp
from jax import lax
from jax.experimental import pallas as pl
from jax.experimental.pallas import tpu as pltpu
from jax.experimental.pallas import tpu_sc as plsc

_NC = 2
_NS = 16
_L = 16
_CH = 80
_ZR = 128


def _pad_rows(n):
    step = 8 * _NS
    return (n + step - 1) // step * step


def _sc_segment_sums(nodes_lo, nodes_hi, edges, senders, receivers):
    n, dh = nodes_lo.shape
    e, de = edges.shape
    ept = e // _NS
    nch = ept // _CH
    assert nch % 2 == 0
    npad = _pad_rows(n)
    rpt = npad // _NS

    mesh = plsc.VectorSubcoreMesh(core_axis_name="c", subcore_axis_name="s")

    @functools.partial(
        pl.kernel,
        out_type=(
            jax.ShapeDtypeStruct((_NC * npad, dh), jnp.float32),
            jax.ShapeDtypeStruct((_NC * npad, de), jnp.float32),
        ),
        mesh=mesh,
        scratch_types=[
            pltpu.VMEM((_CH,), jnp.int32),
            pltpu.VMEM((_CH,), jnp.int32),
            pltpu.VMEM((_CH, dh), jnp.float32),
            pltpu.VMEM((_CH, de), jnp.float32),
            pltpu.VMEM((_ZR, dh), jnp.float32),
            pltpu.VMEM((rpt, de), jnp.float32),
            pltpu.VMEM_SHARED((npad, dh), jnp.float32),
            pltpu.VMEM_SHARED((npad, de), jnp.float32),
            pltpu.SemaphoreType.DMA,
        ],
        compiler_params=pltpu.CompilerParams(use_tc_tiling_on_sc=False),
    )
    def seg(lo_h, hi_h, edges_h, send_h, recv_h, gout_h, eout_h,
            idx_s, idx_r, rows, erows, zn, ze, g_sh, e_sh, sem):
        c = lax.axis_index("c")
        s = lax.axis_index("s")
        zero16 = jnp.zeros((_L,), jnp.float32)
        dsub = dh // _L

        @pl.loop(0, _ZR * dsub)
        def _(i):
            zn[i // dsub, pl.ds((i % dsub) * _L, _L)] = zero16

        @pl.loop(0, rpt)
        def _(i):
            ze[i, :] = zero16

        row0 = s * rpt
        nfull = rpt // _ZR

        @pl.loop(0, nfull)
        def _(i):
            pltpu.sync_copy(zn, g_sh.at[pl.ds(row0 + i * _ZR, _ZR)])

        rem = rpt - nfull * _ZR
        if rem:
            pltpu.sync_copy(zn.at[pl.ds(0, rem)],
                            g_sh.at[pl.ds(row0 + nfull * _ZR, rem)])
        pltpu.sync_copy(ze, e_sh.at[pl.ds(row0, rpt)])

        plsc.subcore_barrier()

        base = s * ept
        ehalf0 = c * (nch // 2)

        @pl.loop(0, nch)
        def _(k):
            off = pl.multiple_of(base + k * _CH, 8)
            pltpu.sync_copy(send_h.at[pl.ds(off, _CH)], idx_s)
            pltpu.sync_copy(recv_h.at[pl.ds(off, _CH)], idx_r)

            @pl.when(c == 0)
            def _():
                pltpu.async_copy(lo_h.at[idx_s], rows, sem).wait()

            @pl.when(c == 1)
            def _():
                pltpu.async_copy(hi_h.at[idx_s], rows, sem).wait()

            pltpu.sync_copy(rows, g_sh.at[idx_r], add=True)

            @pl.when((k >= ehalf0) & (k < ehalf0 + nch // 2))
            def _():
                pltpu.sync_copy(edges_h.at[pl.ds(off, _CH)], erows)
                pltpu.sync_copy(erows, e_sh.at[idx_r], add=True)

        plsc.subcore_barrier()

        out0 = c * npad + row0
        pltpu.sync_copy(g_sh.at[pl.ds(row0, rpt)], gout_h.at[pl.ds(out0, rpt)])
        pltpu.sync_copy(e_sh.at[pl.ds(row0, rpt)], eout_h.at[pl.ds(out0, rpt)])

    return seg(nodes_lo, nodes_hi, edges, senders, receivers)


def _dense_body(g_ref, e_ref, x_ref, a_ref, b_ref, w1_ref, b1_ref,
                w2_ref, b2_ref, s1_ref, o1_ref, s2_ref, o2_ref, out_ref):
    hi = lax.Precision.HIGHEST
    g = jnp.concatenate([g_ref[0], g_ref[1]], axis=-1)
    ea = e_ref[0] + e_ref[1]
    x = x_ref[...]
    agg = (jnp.dot(g, a_ref[...], precision=hi)
           + jnp.dot(ea, b_ref[...], precision=hi))
    agg = jnp.maximum(agg, 0.0) + x
    mean = jnp.mean(agg, axis=-1, keepdims=True)
    ctr = agg - mean
    var = jnp.mean(ctr * ctr, axis=-1, keepdims=True)
    aggn = ctr * lax.rsqrt(var + 1e-6) * s1_ref[...] + o1_ref[...]
    h = jnp.maximum(jnp.dot(aggn, w1_ref[...], precision=hi) + b1_ref[...], 0.0)
    y = jnp.dot(h, w2_ref[...], precision=hi) + b2_ref[...] + aggn
    t = x + y
    mean2 = jnp.mean(t, axis=-1, keepdims=True)
    ctr2 = t - mean2
    var2 = jnp.mean(ctr2 * ctr2, axis=-1, keepdims=True)
    out_ref[...] = ctr2 * lax.rsqrt(var2 + 1e-6) * s2_ref[...] + o2_ref[...]


def _tc_dense(gpart, epart, nodes, a, b, w1, b1, w2, b2, s1, o1, s2, o2):
    n, d = nodes.shape
    de = epart.shape[-1]
    bn = 2000
    grid = n // bn
    full = lambda shape: pl.BlockSpec(shape, lambda i, _s=len(shape): (0,) * _s)
    return pl.pallas_call(
        _dense_body,
        grid=(grid,),
        in_specs=[
            pl.BlockSpec((_NC, bn, d // 2), lambda i: (0, i, 0)),
            pl.BlockSpec((_NC, bn, de), lambda i: (0, i, 0)),
            pl.BlockSpec((bn, d), lambda i: (i, 0)),
            full((d, d)), full((de, d)), full((d, d)), full((1, d)),
            full((d, d)), full((1, d)), full((1, d)), full((1, d)),
            full((1, d)), full((1, d)),
        ],
        out_specs=pl.BlockSpec((bn, d), lambda i: (i, 0)),
        out_shape=jax.ShapeDtypeStruct((n, d), jnp.float32),
    )(gpart, epart, nodes, a, b, w1, b1, w2, b2, s1, o1, s2, o2)


def kernel(nodes, edges, receivers, senders, W_msg, W1, b1, W2, b2,
           ln_aggr_scale, ln_aggr_bias, ln_out_scale, ln_out_bias):
    n, d = nodes.shape
    de = edges.shape[1]
    dh = d // 2
    gcols, epart = _sc_segment_sums(nodes[:, :dh], nodes[:, dh:],
                                    edges, senders, receivers)
    npad = _pad_rows(n)
    r = lambda v: v.reshape(1, d)
    return _tc_dense(
        gcols.reshape(_NC, npad, dh), epart.reshape(_NC, npad, de), nodes,
        W_msg[:d], W_msg[d:], W1, r(b1), W2, r(b2),
        r(ln_aggr_scale), r(ln_aggr_bias), r(ln_out_scale), r(ln_out_bias))


# speedup vs baseline: 5.6170x; 2.0844x over previous
import functools

import jax
import jax.numpy as jnp
from jax import lax
from jax.experimental import pallas as pl
from jax.experimental.pallas import tpu as pltpu
from jax.experimental.pallas import tpu_sc as plsc

_NC = 2
_NS = 16
_L = 16
_CH = 80
_ZR = 128


def _pad_rows(n):
    step = 8 * _NS
    return (n + step - 1) // step * step


def _sc_segment_sums(nodes_lo, nodes_hi, edges, senders, receivers):
    n, dh = nodes_lo.shape
    e, de = edges.shape
    ept = e // _NS
    nch = ept // _CH
    assert nch % 2 == 0
    nche = nch // 2
    npad = _pad_rows(n)
    rpt = npad // _NS
    NB = 2
    NBE = 5
    assert nch % NB == 0 and nche % NBE == 0

    mesh = plsc.VectorSubcoreMesh(core_axis_name="c", subcore_axis_name="s")

    @functools.partial(
        pl.kernel,
        out_type=(
            jax.ShapeDtypeStruct((_NC * npad, dh), jnp.float32),
            jax.ShapeDtypeStruct((_NC * npad, de), jnp.float32),
        ),
        mesh=mesh,
        scratch_types=[
            pltpu.VMEM((nch, _CH), jnp.int32),
            pltpu.VMEM((nch, _CH), jnp.int32),
            [pltpu.VMEM((_CH, dh), jnp.float32)] * NB,
            [pltpu.VMEM((_CH, de), jnp.float32)] * NBE,
            pltpu.VMEM((_ZR, dh), jnp.float32),
            pltpu.VMEM((rpt, de), jnp.float32),
            pltpu.VMEM_SHARED((npad, dh), jnp.float32),
            pltpu.VMEM_SHARED((npad, de), jnp.float32),
            [pltpu.SemaphoreType.DMA] * NB,
            [pltpu.SemaphoreType.DMA] * NBE,
        ],
        compiler_params=pltpu.CompilerParams(use_tc_tiling_on_sc=False),
    )
    def seg(lo_h, hi_h, edges_h, send_h, recv_h, gout_h, eout_h,
            idx_s, idx_r, rows, erows, zn, ze, g_sh, e_sh, gsem, esem):
        c = lax.axis_index("c")
        s = lax.axis_index("s")
        zero16 = jnp.zeros((_L,), jnp.float32)
        dsub = dh // _L

        pltpu.sync_copy(send_h.at[pl.ds(s * nch, nch)], idx_s)
        pltpu.sync_copy(recv_h.at[pl.ds(s * nch, nch)], idx_r)

        @pl.loop(0, _ZR * dsub)
        def _(i):
            zn[i // dsub, pl.ds((i % dsub) * _L, _L)] = zero16

        @pl.loop(0, rpt)
        def _(i):
            ze[i, :] = zero16

        row0 = s * rpt
        nfull = rpt // _ZR

        @pl.loop(0, nfull)
        def _(i):
            pltpu.sync_copy(zn, g_sh.at[pl.ds(row0 + i * _ZR, _ZR)])

        rem = rpt - nfull * _ZR
        if rem:
            pltpu.sync_copy(zn.at[pl.ds(0, rem)],
                            g_sh.at[pl.ds(row0 + nfull * _ZR, rem)])
        pltpu.sync_copy(ze, e_sh.at[pl.ds(row0, rpt)])

        plsc.subcore_barrier()

        def gtable(then):
            @pl.when(c == 0)
            def _():
                then(lo_h)

            @pl.when(c == 1)
            def _():
                then(hi_h)

        for b in range(NB):
            gtable(lambda t, b=b: pltpu.async_copy(
                t.at[idx_s.at[b]], rows[b], gsem[b]))

        @pl.loop(0, nch, step=NB)
        def _(k0):
            for b in range(NB):
                k = k0 + b
                pltpu.make_async_copy(lo_h.at[idx_s.at[k]], rows[b],
                                      gsem[b]).wait()
                pltpu.sync_copy(rows[b], g_sh.at[idx_r.at[k]], add=True)

                @pl.when(k + NB < nch)
                def _():
                    gtable(lambda t, b=b: pltpu.async_copy(
                        t.at[idx_s.at[k + NB]], rows[b], gsem[b]))

        base = s * ept
        eh0 = c * nche
        for b in range(NBE):
            pltpu.async_copy(
                edges_h.at[pl.ds(base + (eh0 + b) * _CH, _CH)],
                erows[b], esem[b])

        @pl.loop(0, nche, step=NBE)
        def _(k0):
            for b in range(NBE):
                k = eh0 + k0 + b
                pltpu.make_async_copy(
                    edges_h.at[pl.ds(base + k * _CH, _CH)],
                    erows[b], esem[b]).wait()
                pltpu.sync_copy(erows[b], e_sh.at[idx_r.at[k]], add=True)

                @pl.when(k0 + b + NBE < nche)
                def _():
                    pltpu.async_copy(
                        edges_h.at[pl.ds(base + (k + NBE) * _CH, _CH)],
                        erows[b], esem[b])

        plsc.subcore_barrier()

        out0 = c * npad + row0
        pltpu.sync_copy(g_sh.at[pl.ds(row0, rpt)], gout_h.at[pl.ds(out0, rpt)])
        pltpu.sync_copy(e_sh.at[pl.ds(row0, rpt)], eout_h.at[pl.ds(out0, rpt)])

    return seg(nodes_lo, nodes_hi, edges,
               senders.reshape(e // _CH, _CH), receivers.reshape(e // _CH, _CH))


def _dense_body(g_ref, e_ref, x_ref, a_ref, b_ref, w1_ref, b1_ref,
                w2_ref, b2_ref, s1_ref, o1_ref, s2_ref, o2_ref, out_ref):
    hi = lax.Precision.HIGHEST
    g = jnp.concatenate([g_ref[0], g_ref[1]], axis=-1)
    ea = e_ref[0] + e_ref[1]
    x = x_ref[...]
    agg = (jnp.dot(g, a_ref[...], precision=hi)
           + jnp.dot(ea, b_ref[...], precision=hi))
    agg = jnp.maximum(agg, 0.0) + x
    mean = jnp.mean(agg, axis=-1, keepdims=True)
    ctr = agg - mean
    var = jnp.mean(ctr * ctr, axis=-1, keepdims=True)
    aggn = ctr * lax.rsqrt(var + 1e-6) * s1_ref[...] + o1_ref[...]
    h = jnp.maximum(jnp.dot(aggn, w1_ref[...], precision=hi) + b1_ref[...], 0.0)
    y = jnp.dot(h, w2_ref[...], precision=hi) + b2_ref[...] + aggn
    t = x + y
    mean2 = jnp.mean(t, axis=-1, keepdims=True)
    ctr2 = t - mean2
    var2 = jnp.mean(ctr2 * ctr2, axis=-1, keepdims=True)
    out_ref[...] = ctr2 * lax.rsqrt(var2 + 1e-6) * s2_ref[...] + o2_ref[...]


def _tc_dense(gpart, epart, nodes, a, b, w1, b1, w2, b2, s1, o1, s2, o2):
    n, d = nodes.shape
    de = epart.shape[-1]
    bn = 2000
    grid = n // bn
    full = lambda shape: pl.BlockSpec(shape, lambda i, _s=len(shape): (0,) * _s)
    return pl.pallas_call(
        _dense_body,
        grid=(grid,),
        in_specs=[
            pl.BlockSpec((_NC, bn, d // 2), lambda i: (0, i, 0)),
            pl.BlockSpec((_NC, bn, de), lambda i: (0, i, 0)),
            pl.BlockSpec((bn, d), lambda i: (i, 0)),
            full((d, d)), full((de, d)), full((d, d)), full((1, d)),
            full((d, d)), full((1, d)), full((1, d)), full((1, d)),
            full((1, d)), full((1, d)),
        ],
        out_specs=pl.BlockSpec((bn, d), lambda i: (i, 0)),
        out_shape=jax.ShapeDtypeStruct((n, d), jnp.float32),
    )(gpart, epart, nodes, a, b, w1, b1, w2, b2, s1, o1, s2, o2)


def kernel(nodes, edges, receivers, senders, W_msg, W1, b1, W2, b2,
           ln_aggr_scale, ln_aggr_bias, ln_out_scale, ln_out_bias):
    n, d = nodes.shape
    de = edges.shape[1]
    dh = d // 2
    gcols, epart = _sc_segment_sums(nodes[:, :dh], nodes[:, dh:],
                                    edges, senders, receivers)
    npad = _pad_rows(n)
    r = lambda v: v.reshape(1, d)
    return _tc_dense(
        gcols.reshape(_NC, npad, dh), epart.reshape(_NC, npad, de), nodes,
        W_msg[:d], W_msg[d:], W1, r(b1), W2, r(b2),
        r(ln_aggr_scale), r(ln_aggr_bias), r(ln_out_scale), r(ln_out_bias))
